# baseline (speedup 1.0000x reference)
import functools

import jax
import jax.numpy as jnp
from jax import lax
from jax.experimental import pallas as pl
from jax.experimental.pallas import tpu as pltpu
from jax.experimental.pallas import tpu_sc as plsc

EMBED_DIM = 128
NC = 2
NS = 16
NW = NC * NS
CHUNK = 128
NBUF = 6
LEAD = 2


@functools.lru_cache(maxsize=None)
def _make_kernel(B: int):
    assert B % (NW * CHUNK) == 0
    n_chunks = B // (NW * CHUNK)
    assert n_chunks > NBUF
    mesh = plsc.VectorSubcoreMesh(core_axis_name="c", subcore_axis_name="s")

    @functools.partial(
        pl.kernel,
        mesh=mesh,
        out_type=jax.ShapeDtypeStruct((B, EMBED_DIM), jnp.float32),
        scratch_types=[
            pltpu.VMEM((n_chunks, CHUNK), jnp.int32),
            pltpu.VMEM((NBUF, CHUNK, EMBED_DIM), jnp.float32),
            pltpu.VMEM_SHARED((1000, EMBED_DIM), jnp.float32),
            pltpu.SemaphoreType.DMA((NBUF,)),
            pltpu.SemaphoreType.DMA((NBUF,)),
        ],
    )
    def k(idx_hbm, table_hbm, out_hbm, idx_v, rows_v, table_sp, gsem, ssem):
        wid = lax.axis_index("s") * NC + lax.axis_index("c")
        @pl.when(lax.axis_index("s") == 0)
        def _():
            pltpu.sync_copy(table_hbm, table_sp)

        pltpu.sync_copy(idx_hbm.at[wid], idx_v)
        plsc.subcore_barrier()
        base = wid * n_chunks

        def fire_gather(j):
            b = j % NBUF
            pltpu.async_copy(table_sp.at[idx_v.at[j]], rows_v.at[b], gsem.at[b])

        def drain_gather(j):
            b = j % NBUF
            pltpu.make_async_copy(
                table_sp.at[idx_v.at[j]], rows_v.at[b], gsem.at[b]
            ).wait()

        def fire_scatter(j):
            b = j % NBUF
            pltpu.async_copy(
                rows_v.at[b], out_hbm.at[pl.ds((base + j) * CHUNK, CHUNK)], ssem.at[b]
            )

        def drain_scatter(j):
            b = j % NBUF
            pltpu.make_async_copy(
                rows_v.at[b], out_hbm.at[pl.ds((base + j) * CHUNK, CHUNK)], ssem.at[b]
            ).wait()

        for j0 in range(LEAD):
            fire_gather(j0)

        def body(j, carry):
            drain_gather(j)
            fire_scatter(j)
            @pl.when(j >= NBUF - LEAD)
            def _():
                drain_scatter(j - (NBUF - LEAD))

            @pl.when(j + LEAD < n_chunks)
            def _():
                fire_gather(j + LEAD)

            return carry

        lax.fori_loop(0, n_chunks, body, 0)
        for t in range(NBUF - LEAD):
            drain_scatter(n_chunks - (NBUF - LEAD) + t)

    return k


@jax.jit
def kernel(indices, species_table):
    B, H = indices.shape
    n = B * H
    idx3d = indices.T.reshape(NW, n // (NW * CHUNK), CHUNK).astype(jnp.int32)
    out = _make_kernel(n)(idx3d, species_table)
    return out.reshape(H, B, EMBED_DIM).transpose(1, 0, 2)


# speedup vs baseline: 1.0238x; 1.0238x over previous
import functools

import jax
import jax.numpy as jnp
from jax import lax
from jax.experimental import pallas as pl
from jax.experimental.pallas import tpu as pltpu
from jax.experimental.pallas import tpu_sc as plsc

EMBED_DIM = 128
NC = 2
NS = 16
NW = NC * NS
CHUNK = 64
NBUF = 12
LEAD = 6


@functools.lru_cache(maxsize=None)
def _make_kernel(B: int):
    assert B % (NW * CHUNK) == 0
    n_chunks = B // (NW * CHUNK)
    assert n_chunks > NBUF
    mesh = plsc.VectorSubcoreMesh(core_axis_name="c", subcore_axis_name="s")

    @functools.partial(
        pl.kernel,
        mesh=mesh,
        out_type=jax.ShapeDtypeStruct((B, EMBED_DIM), jnp.float32),
        scratch_types=[
            pltpu.VMEM((n_chunks, CHUNK), jnp.int32),
            pltpu.VMEM((NBUF, CHUNK, EMBED_DIM), jnp.float32),
            pltpu.VMEM_SHARED((1000, EMBED_DIM), jnp.float32),
            pltpu.SemaphoreType.DMA((NBUF,)),
            pltpu.SemaphoreType.DMA((NBUF,)),
        ],
    )
    def k(idx_hbm, table_hbm, out_hbm, idx_v, rows_v, table_sp, gsem, ssem):
        wid = lax.axis_index("s") * NC + lax.axis_index("c")
        @pl.when(lax.axis_index("s") == 0)
        def _():
            pltpu.sync_copy(table_hbm, table_sp)

        pltpu.sync_copy(idx_hbm.at[wid], idx_v)
        plsc.subcore_barrier()
        base = wid * n_chunks

        def fire_gather(j):
            b = j % NBUF
            pltpu.async_copy(table_sp.at[idx_v.at[j]], rows_v.at[b], gsem.at[b])

        def drain_gather(j):
            b = j % NBUF
            pltpu.make_async_copy(
                table_sp.at[idx_v.at[j]], rows_v.at[b], gsem.at[b]
            ).wait()

        def fire_scatter(j):
            b = j % NBUF
            pltpu.async_copy(
                rows_v.at[b], out_hbm.at[pl.ds((base + j) * CHUNK, CHUNK)], ssem.at[b]
            )

        def drain_scatter(j):
            b = j % NBUF
            pltpu.make_async_copy(
                rows_v.at[b], out_hbm.at[pl.ds((base + j) * CHUNK, CHUNK)], ssem.at[b]
            ).wait()

        for j0 in range(LEAD):
            fire_gather(j0)

        def body(j, carry):
            drain_gather(j)
            fire_scatter(j)
            @pl.when(j >= NBUF - LEAD)
            def _():
                drain_scatter(j - (NBUF - LEAD))

            @pl.when(j + LEAD < n_chunks)
            def _():
                fire_gather(j + LEAD)

            return carry

        lax.fori_loop(0, n_chunks, body, 0)
        for t in range(NBUF - LEAD):
            drain_scatter(n_chunks - (NBUF - LEAD) + t)

    return k


@jax.jit
def kernel(indices, species_table):
    B, H = indices.shape
    n = B * H
    idx3d = indices.T.reshape(NW, n // (NW * CHUNK), CHUNK).astype(jnp.int32)
    out = _make_kernel(n)(idx3d, species_table)
    return out.reshape(H, B, EMBED_DIM).transpose(1, 0, 2)
